# Initial kernel scaffold; baseline (speedup 1.0000x reference)
import jax
import jax.numpy as jnp
from jax.experimental import pallas as pl


def kernel(tokens, emb_table):
    raise NotImplementedError("write your pallas kernel here")


# speedup vs baseline: 2.9677x; 2.9677x over previous
import math

import jax
import jax.numpy as jnp
from jax import lax
from jax.experimental import pallas as pl
from jax.experimental.pallas import tpu as pltpu

_TILE = 512
_UNROLL = 8


def _round_up(x: int, m: int) -> int:
    return ((x + m - 1) // m) * m


def _make_body(tile: int, unroll: int, scale: float):
    def _body(ids_ref, tbl_ref, out_ref, scratch):
        def gather_one(mi):
            tok = ids_ref[0, 0, mi]
            base = pl.multiple_of((tok >> 3) << 3, 8)
            chunk = tbl_ref[pl.ds(base, 8), :].astype(jnp.float32)
            rolled = pltpu.roll(chunk, -(tok & 7), axis=0)
            scratch[pl.ds(mi, 1), :] = rolled[0:1, :]

        @pl.loop(0, tile // unroll)
        def _(k):
            for j in range(unroll):
                gather_one(k * unroll + j)

        out_ref[...] = (scratch[...] * scale).astype(out_ref.dtype)

    return _body


def kernel(tokens, emb_table):
    vocab, emb = emb_table.shape
    scale = float(math.sqrt(emb))
    out_dtype = emb_table.dtype

    flat = jnp.clip(tokens.reshape(-1).astype(jnp.int32), 0, vocab - 1)
    n_tok = int(flat.shape[0])

    tile = min(_TILE, _round_up(n_tok, _UNROLL))
    n_pad = _round_up(n_tok, tile)
    ids = jnp.pad(flat, (0, n_pad - n_tok))
    n_blocks = n_pad // tile
    ids3d = ids.reshape(n_blocks, 1, tile)

    v_pad = _round_up(vocab, 8)
    tbl = emb_table
    if v_pad != vocab:
        tbl = jnp.pad(emb_table, ((0, v_pad - vocab), (0, 0)))

    itemsize = jnp.dtype(out_dtype).itemsize
    vmem_limit = int(2 * v_pad * emb * itemsize
                     + 2 * tile * emb * itemsize
                     + tile * emb * 4
                     + (4 << 20))

    out_flat = pl.pallas_call(
        _make_body(tile, _UNROLL, scale),
        out_shape=jax.ShapeDtypeStruct((n_pad, emb), out_dtype),
        grid=(n_blocks,),
        in_specs=[
            pl.BlockSpec((1, 1, tile), lambda i: (i, 0, 0),
                         memory_space=pltpu.SMEM),
            pl.BlockSpec((v_pad, emb), lambda i: (0, 0)),
        ],
        out_specs=pl.BlockSpec((tile, emb), lambda i: (i, 0)),
        scratch_shapes=[pltpu.VMEM((tile, emb), jnp.float32)],
        compiler_params=pltpu.CompilerParams(
            dimension_semantics=("parallel",),
            vmem_limit_bytes=min(vmem_limit, 128 << 20),
        ),
    )(ids3d, tbl)

    return out_flat[:n_tok].reshape(tokens.shape + (emb,))
